# baseline (speedup 1.0000x reference)
import jax
import jax.numpy as jnp
from jax.experimental import pallas as pl
from jax.experimental.pallas import tpu as pltpu

BM = 1000
BK = 2048
NPAD = 10240


def _gcn_kernel(x_ref, adj_ref, w_ref, b_ref, out_ref,
                support_ref, acc_ref):
    n = x_ref.shape[0]
    k = pl.program_id(1)
    nk = pl.num_programs(1)

    @pl.when(jnp.logical_and(pl.program_id(0) == 0, k == 0))
    def _():
        support_ref[...] = jnp.zeros_like(support_ref)
        support_ref[pl.ds(0, n), :] = jnp.dot(
            x_ref[...], w_ref[...], preferred_element_type=jnp.float32
        )

    a = adj_ref[...]

    @pl.when(k == nk - 1)
    def _():
        cols = jax.lax.broadcasted_iota(jnp.int32, (1, BK), 1) + k * BK
        acc_ref[...] += jnp.dot(
            jnp.where(cols < n, a, 0.0),
            support_ref[pl.ds(k * BK, BK), :],
            preferred_element_type=jnp.float32,
        )
        out_ref[...] = jnp.maximum(acc_ref[...] + b_ref[...], 0.0)

    @pl.when(k < nk - 1)
    def _():
        part = jnp.dot(
            a,
            support_ref[pl.ds(k * BK, BK), :],
            preferred_element_type=jnp.float32,
        )

        @pl.when(k == 0)
        def _():
            acc_ref[...] = part

        @pl.when(k > 0)
        def _():
            acc_ref[...] += part


@jax.jit
def kernel(x, adj, W, b):
    n, nfeat = x.shape
    nhid = W.shape[1]
    b2 = b.reshape(1, nhid)
    grid = (n // BM, pl.cdiv(n, BK))
    return pl.pallas_call(
        _gcn_kernel,
        grid=grid,
        in_specs=[
            pl.BlockSpec((n, nfeat), lambda i, k: (0, 0)),
            pl.BlockSpec((BM, BK), lambda i, k: (i, k)),
            pl.BlockSpec((nfeat, nhid), lambda i, k: (0, 0)),
            pl.BlockSpec((1, nhid), lambda i, k: (0, 0)),
        ],
        out_specs=pl.BlockSpec((BM, nhid), lambda i, k: (i, 0)),
        out_shape=jax.ShapeDtypeStruct((n, nhid), jnp.float32),
        scratch_shapes=[
            pltpu.VMEM((NPAD, nhid), jnp.float32),
            pltpu.VMEM((BM, nhid), jnp.float32),
        ],
        compiler_params=pltpu.CompilerParams(
            dimension_semantics=("arbitrary", "arbitrary"),
        ),
    )(x, adj, W, b2)


# speedup vs baseline: 1.1200x; 1.1200x over previous
import jax
import jax.numpy as jnp
from jax.experimental import pallas as pl
from jax.experimental.pallas import tpu as pltpu

BM = 400


def _gcn_kernel(x_ref, adj_ref, w_ref, b_ref, out_ref, support_ref):
    @pl.when(pl.program_id(0) == 0)
    def _():
        support_ref[...] = jnp.dot(
            x_ref[...], w_ref[...], preferred_element_type=jnp.float32
        )

    acc = jnp.dot(
        adj_ref[...], support_ref[...], preferred_element_type=jnp.float32
    )
    out_ref[...] = jnp.maximum(acc + b_ref[...], 0.0)


@jax.jit
def kernel(x, adj, W, b):
    n, nfeat = x.shape
    nhid = W.shape[1]
    b2 = b.reshape(1, nhid)
    grid = (n // BM,)
    return pl.pallas_call(
        _gcn_kernel,
        grid=grid,
        in_specs=[
            pl.BlockSpec((n, nfeat), lambda i: (0, 0)),
            pl.BlockSpec((BM, n), lambda i: (i, 0)),
            pl.BlockSpec((nfeat, nhid), lambda i: (0, 0)),
            pl.BlockSpec((1, nhid), lambda i: (0, 0)),
        ],
        out_specs=pl.BlockSpec((BM, nhid), lambda i: (i, 0)),
        out_shape=jax.ShapeDtypeStruct((n, nhid), jnp.float32),
        scratch_shapes=[pltpu.VMEM((n, nhid), jnp.float32)],
        compiler_params=pltpu.CompilerParams(
            dimension_semantics=("arbitrary",),
        ),
    )(x, adj, W, b2)
